# baseline (speedup 1.0000x reference)
import functools

import jax
import jax.numpy as jnp
from jax import lax
from jax.experimental import pallas as pl
from jax.experimental.pallas import tpu as pltpu
from jax.experimental.pallas import tpu_sc as plsc

B, IN_DIM, HID, EMB, K, L = 8, 128, 768, 256, 8192, 2048
T = L // 4
NTOK = B * T
F32 = jnp.float32


def _mm(a, b):
    return jnp.dot(a, b, preferred_element_type=F32)


def _stride2_body(x0_ref, x1_ref, w_ref, b_ref, o_ref, *, kdim):
    w0 = w_ref[0:kdim, :]
    w1 = w_ref[kdim:2 * kdim, :]
    acc = _mm(x0_ref[0], w0) + _mm(x1_ref[0], w1) + b_ref[0, :][None, :]
    o_ref[0] = jnp.maximum(acc, 0.0)


def _stride2_conv(x0, x1, w, b, n_out, c_out):
    kdim = x0.shape[-1]
    return pl.pallas_call(
        functools.partial(_stride2_body, kdim=kdim),
        grid=(B,),
        in_specs=[
            pl.BlockSpec((1, n_out, kdim), lambda i: (i, 0, 0)),
            pl.BlockSpec((1, n_out, kdim), lambda i: (i, 0, 0)),
            pl.BlockSpec((2 * kdim, c_out), lambda i: (0, 0)),
            pl.BlockSpec((1, c_out), lambda i: (0, 0)),
        ],
        out_specs=pl.BlockSpec((1, n_out, c_out), lambda i: (i, 0, 0)),
        out_shape=jax.ShapeDtypeStruct((B, n_out, c_out), F32),
    )(x0, x1, w, b)


def _enc3_body(h_ref, w_ref, b_ref, o_ref):
    acc = b_ref[0, :][None, :]
    for j in range(3):
        acc = acc + _mm(h_ref[0, j:T + j, :], w_ref[j])
    o_ref[0] = acc


def _enc3(hpad, w, b):
    return pl.pallas_call(
        _enc3_body,
        grid=(B,),
        in_specs=[
            pl.BlockSpec((1, T + 2, HID), lambda i: (i, 0, 0)),
            pl.BlockSpec((3, HID, EMB), lambda i: (0, 0, 0)),
            pl.BlockSpec((1, EMB), lambda i: (0, 0)),
        ],
        out_specs=pl.BlockSpec((1, T, EMB), lambda i: (i, 0, 0)),
        out_shape=jax.ShapeDtypeStruct((B, T, EMB), F32),
    )(hpad, w, b)


_KC = 2048


def _quant_body(z_ref, cbt_ref, idx_ref, cnt_ref):
    pid = pl.program_id(0)
    zb = z_ref[0]
    zn = jnp.sum(zb * zb, axis=1, keepdims=True)
    best_v = jnp.full((T, 1), jnp.inf, F32)
    best_i = jnp.zeros((T, 1), jnp.int32)
    for kc in range(K // _KC):
        cbt = cbt_ref[:, kc * _KC:(kc + 1) * _KC]
        cn = jnp.sum(cbt * cbt, axis=0, keepdims=True)
        d = zn + cn - 2.0 * _mm(zb, cbt)
        m = jnp.min(d, axis=1, keepdims=True)
        ids = lax.broadcasted_iota(jnp.int32, (T, _KC), 1) + kc * _KC
        i = jnp.min(jnp.where(d == m, ids, K), axis=1, keepdims=True)
        upd = m < best_v
        best_i = jnp.where(upd, i, best_i)
        best_v = jnp.where(upd, m, best_v)
    idx_ref[0, 0, :] = best_i[:, 0]
    for kc in range(K // _KC):
        ids = lax.broadcasted_iota(jnp.int32, (1, _KC), 1) + kc * _KC
        cnt = jnp.sum((best_i == ids).astype(F32), axis=0)
        sl = pl.ds(kc * _KC, _KC)

        @pl.when(pid == 0)
        def _():
            cnt_ref[0, sl] = cnt

        @pl.when(pid != 0)
        def _():
            cnt_ref[0, sl] = cnt_ref[0, sl] + cnt


def _quantize(z, cbt):
    return pl.pallas_call(
        _quant_body,
        grid=(B,),
        in_specs=[
            pl.BlockSpec((1, T, EMB), lambda i: (i, 0, 0)),
            pl.BlockSpec((EMB, K), lambda i: (0, 0)),
        ],
        out_specs=[
            pl.BlockSpec((1, 1, T), lambda i: (i, 0, 0)),
            pl.BlockSpec((1, K), lambda i: (0, 0)),
        ],
        out_shape=[
            jax.ShapeDtypeStruct((B, 1, T), jnp.int32),
            jax.ShapeDtypeStruct((1, K), F32),
        ],
    )(z, cbt)


def _sc_gather(codebook, idx_flat):
    info = plsc.get_sparse_core_info()
    nc, ns = info.num_cores, info.num_subcores
    nw = nc * ns
    bpw = NTOK // nw
    mesh = plsc.VectorSubcoreMesh(core_axis_name="c", subcore_axis_name="s")

    @functools.partial(
        pl.kernel,
        mesh=mesh,
        out_type=jax.ShapeDtypeStruct((NTOK, EMB), F32),
        scratch_types=[
            pltpu.VMEM((bpw,), jnp.int32),
            pltpu.VMEM((bpw, EMB), F32),
            pltpu.SemaphoreType.DMA,
        ],
    )
    def gk(cb_hbm, idx_hbm, out_hbm, idx_v, rows_v, sem):
        wid = lax.axis_index("s") * nc + lax.axis_index("c")
        base = wid * bpw
        pltpu.sync_copy(idx_hbm.at[pl.ds(base, bpw)], idx_v)
        pltpu.async_copy(cb_hbm.at[idx_v], rows_v, sem).wait()
        pltpu.sync_copy(rows_v, out_hbm.at[pl.ds(base, bpw)])

    return gk(codebook, idx_flat)


def _dec1_body(qpad_ref, z_ref, w_ref, b_ref, o_ref, sse_ref):
    acc = b_ref[0, :][None, :]
    for j in range(3):
        acc = acc + _mm(qpad_ref[0, j:T + j, :], w_ref[j])
    o_ref[0] = jnp.maximum(acc, 0.0)
    diff = qpad_ref[0, 1:T + 1, :] - z_ref[0]
    sse_ref[0, 0, :] = jnp.full((128,), jnp.sum(diff * diff), F32)


def _dec1(qpad, z, w, b):
    return pl.pallas_call(
        _dec1_body,
        grid=(B,),
        in_specs=[
            pl.BlockSpec((1, T + 2, EMB), lambda i: (i, 0, 0)),
            pl.BlockSpec((1, T, EMB), lambda i: (i, 0, 0)),
            pl.BlockSpec((3, EMB, HID), lambda i: (0, 0, 0)),
            pl.BlockSpec((1, HID), lambda i: (0, 0)),
        ],
        out_specs=[
            pl.BlockSpec((1, T, HID), lambda i: (i, 0, 0)),
            pl.BlockSpec((1, 1, 128), lambda i: (i, 0, 0)),
        ],
        out_shape=[
            jax.ShapeDtypeStruct((B, T, HID), F32),
            jax.ShapeDtypeStruct((B, 1, 128), F32),
        ],
    )(qpad, z, w, b)


def _convt_body(inpad_ref, w_ref, b_ref, e_ref, o_ref, *, n, relu):
    bb = b_ref[0, :][None, :]
    ev = _mm(inpad_ref[0, 1:n + 1, :], w_ref[1]) + _mm(inpad_ref[0, 0:n, :], w_ref[3]) + bb
    od = _mm(inpad_ref[0, 2:n + 2, :], w_ref[0]) + _mm(inpad_ref[0, 1:n + 1, :], w_ref[2]) + bb
    if relu:
        ev = jnp.maximum(ev, 0.0)
        od = jnp.maximum(od, 0.0)
    e_ref[0] = ev
    o_ref[0] = od


def _convt(inpad, w, b, n, c_in, c_out, relu):
    return pl.pallas_call(
        functools.partial(_convt_body, n=n, relu=relu),
        grid=(B,),
        in_specs=[
            pl.BlockSpec((1, n + 2, c_in), lambda i: (i, 0, 0)),
            pl.BlockSpec((4, c_in, c_out), lambda i: (0, 0, 0)),
            pl.BlockSpec((1, c_out), lambda i: (0, 0)),
        ],
        out_specs=[
            pl.BlockSpec((1, n, c_out), lambda i: (i, 0, 0)),
            pl.BlockSpec((1, n, c_out), lambda i: (i, 0, 0)),
        ],
        out_shape=[
            jax.ShapeDtypeStruct((B, n, c_out), F32),
            jax.ShapeDtypeStruct((B, n, c_out), F32),
        ],
    )(inpad, w, b)


def _final_body(cnt_ref, sse_ref, loss_ref, perp_ref):
    p = cnt_ref[0, :] * (1.0 / NTOK)
    ent = -jnp.sum(p * jnp.log(p + 1e-10))
    perp_ref[...] = jnp.full((1, 1), jnp.exp(ent), F32)
    total = jnp.sum(sse_ref[:, 0, 0:1])
    loss_ref[...] = jnp.full((1, 1), 1.25 * total / (NTOK * EMB), F32)


def _final(cnt, sse):
    return pl.pallas_call(
        _final_body,
        in_specs=[
            pl.BlockSpec((1, K), lambda: (0, 0)),
            pl.BlockSpec((B, 1, 128), lambda: (0, 0, 0)),
        ],
        out_specs=[
            pl.BlockSpec((1, 1), lambda: (0, 0)),
            pl.BlockSpec((1, 1), lambda: (0, 0)),
        ],
        out_shape=[
            jax.ShapeDtypeStruct((1, 1), F32),
            jax.ShapeDtypeStruct((1, 1), F32),
        ],
    )(cnt, sse)


def kernel(x, enc_w1, enc_b1, enc_w2, enc_b2, enc_w3, enc_b3, codebook,
           dec_w1, dec_b1, dec_w2, dec_b2, dec_w3, dec_b3):
    xpad = jnp.pad(jnp.transpose(x, (0, 2, 1)), ((0, 0), (1, 1), (0, 0)))
    x0 = xpad[:, 0:L, :].reshape(B, L // 2, 2 * IN_DIM)
    x1 = xpad[:, 2:L + 2, :].reshape(B, L // 2, 2 * IN_DIM)
    we1 = enc_w1.transpose(2, 1, 0).reshape(4 * IN_DIM, HID)
    h1 = _stride2_conv(x0, x1, we1, enc_b1.reshape(1, HID), L // 2, HID)

    h1pad = jnp.pad(h1, ((0, 0), (1, 1), (0, 0)))
    h0 = h1pad[:, 0:L // 2, :].reshape(B, T, 2 * HID)
    h1s = h1pad[:, 2:L // 2 + 2, :].reshape(B, T, 2 * HID)
    we2 = enc_w2.transpose(2, 1, 0).reshape(4 * HID, HID)
    h2 = _stride2_conv(h0, h1s, we2, enc_b2.reshape(1, HID), T, HID)

    h2pad = jnp.pad(h2, ((0, 0), (1, 1), (0, 0)))
    we3 = enc_w3.transpose(2, 1, 0)
    z = _enc3(h2pad, we3, enc_b3.reshape(1, EMB))

    idx3, counts = _quantize(z, codebook.T)
    q_flat = _sc_gather(codebook, idx3.reshape(NTOK))
    q = q_flat.reshape(B, T, EMB)

    qpad = jnp.pad(q, ((0, 0), (1, 1), (0, 0)))
    wd1 = dec_w1.transpose(2, 1, 0)
    d1, sse = _dec1(qpad, z, wd1, dec_b1.reshape(1, HID))

    d1pad = jnp.pad(d1, ((0, 0), (1, 1), (0, 0)))
    wd2 = dec_w2.transpose(2, 0, 1)
    e2, o2 = _convt(d1pad, wd2, dec_b2.reshape(1, HID), T, HID, HID, True)
    d2 = jnp.stack([e2, o2], axis=2).reshape(B, L // 2, HID)

    d2pad = jnp.pad(d2, ((0, 0), (1, 1), (0, 0)))
    wd3 = dec_w3.transpose(2, 0, 1)
    e3, o3 = _convt(d2pad, wd3, dec_b3.reshape(1, IN_DIM), L // 2, HID, IN_DIM, False)
    x_recon = jnp.stack([e3, o3], axis=2).reshape(B, L, IN_DIM).transpose(0, 2, 1)

    loss, perp = _final(counts, sse)
    return (loss.reshape(()), x_recon, perp.reshape(()))


# speedup vs baseline: 1.9115x; 1.9115x over previous
import functools

import jax
import jax.numpy as jnp
from jax import lax
from jax.experimental import pallas as pl
from jax.experimental.pallas import tpu as pltpu
from jax.experimental.pallas import tpu_sc as plsc

B, IN_DIM, HID, EMB, K, L = 8, 128, 768, 256, 8192, 2048
T = L // 4
NTOK = B * T
F32 = jnp.float32
_KC = 2048


def _mm(a, b):
    return jnp.dot(a, b, preferred_element_type=F32)


def _encq_body(xq0_ref, xq1_ref, w1_ref, b1_ref, w2_ref, b2_ref, w3_ref,
               b3_ref, cbt_ref, z_ref, idx_ref, cnt_ref, ep_ref, op_ref, hp_ref):
    pid = pl.program_id(0)
    zrow = jnp.zeros((1, HID), F32)
    b1 = b1_ref[0, :][None, :]
    he = jnp.maximum(_mm(xq0_ref[0], w1_ref[...]) + b1, 0.0)
    ho = jnp.maximum(_mm(xq1_ref[0], w1_ref[...]) + b1, 0.0)
    ep_ref[0:1, :] = zrow
    ep_ref[T + 1:T + 2, :] = zrow
    ep_ref[1:T + 1, :] = he
    op_ref[0:1, :] = zrow
    op_ref[T + 1:T + 2, :] = zrow
    op_ref[1:T + 1, :] = ho
    b2 = b2_ref[0, :][None, :]
    h2 = (_mm(op_ref[0:T, :], w2_ref[0]) + _mm(ep_ref[1:T + 1, :], w2_ref[1])
          + _mm(op_ref[1:T + 1, :], w2_ref[2]) + _mm(ep_ref[2:T + 2, :], w2_ref[3]) + b2)
    h2 = jnp.maximum(h2, 0.0)
    hp_ref[0:1, :] = zrow
    hp_ref[T + 1:T + 2, :] = zrow
    hp_ref[1:T + 1, :] = h2
    zb = b3_ref[0, :][None, :]
    for j in range(3):
        zb = zb + _mm(hp_ref[j:T + j, :], w3_ref[j])
    z_ref[0] = zb
    zn = jnp.sum(zb * zb, axis=1, keepdims=True)
    best_v = jnp.full((T, 1), jnp.inf, F32)
    best_i = jnp.zeros((T, 1), jnp.int32)
    for kc in range(K // _KC):
        cbt = cbt_ref[:, kc * _KC:(kc + 1) * _KC]
        cn = jnp.sum(cbt * cbt, axis=0, keepdims=True)
        d = zn + cn - 2.0 * _mm(zb, cbt)
        m = jnp.min(d, axis=1, keepdims=True)
        ids = lax.broadcasted_iota(jnp.int32, (T, _KC), 1) + kc * _KC
        i = jnp.min(jnp.where(d == m, ids, K), axis=1, keepdims=True)
        upd = m < best_v
        best_i = jnp.where(upd, i, best_i)
        best_v = jnp.where(upd, m, best_v)
    idx_ref[0, 0, :] = best_i[:, 0]
    for kc in range(K // _KC):
        ids = lax.broadcasted_iota(jnp.int32, (1, _KC), 1) + kc * _KC
        cnt = jnp.sum((best_i == ids).astype(F32), axis=0)
        sl = pl.ds(kc * _KC, _KC)

        @pl.when(pid == 0)
        def _():
            cnt_ref[0, sl] = cnt

        @pl.when(pid != 0)
        def _():
            cnt_ref[0, sl] = cnt_ref[0, sl] + cnt


def _encq(xq0, xq1, w1, b1, w2, b2, w3, b3, cbt):
    return pl.pallas_call(
        _encq_body,
        grid=(B,),
        in_specs=[
            pl.BlockSpec((1, T, 4 * IN_DIM), lambda i: (i, 0, 0)),
            pl.BlockSpec((1, T, 4 * IN_DIM), lambda i: (i, 0, 0)),
            pl.BlockSpec((4 * IN_DIM, HID), lambda i: (0, 0)),
            pl.BlockSpec((1, HID), lambda i: (0, 0)),
            pl.BlockSpec((4, HID, HID), lambda i: (0, 0, 0)),
            pl.BlockSpec((1, HID), lambda i: (0, 0)),
            pl.BlockSpec((3, HID, EMB), lambda i: (0, 0, 0)),
            pl.BlockSpec((1, EMB), lambda i: (0, 0)),
            pl.BlockSpec((EMB, K), lambda i: (0, 0)),
        ],
        out_specs=[
            pl.BlockSpec((1, T, EMB), lambda i: (i, 0, 0)),
            pl.BlockSpec((1, 1, T), lambda i: (i, 0, 0)),
            pl.BlockSpec((1, K), lambda i: (0, 0)),
        ],
        out_shape=[
            jax.ShapeDtypeStruct((B, T, EMB), F32),
            jax.ShapeDtypeStruct((B, 1, T), jnp.int32),
            jax.ShapeDtypeStruct((1, K), F32),
        ],
        scratch_shapes=[
            pltpu.VMEM((T + 2, HID), F32),
            pltpu.VMEM((T + 2, HID), F32),
            pltpu.VMEM((T + 2, HID), F32),
        ],
    )(xq0, xq1, w1, b1, w2, b2, w3, b3, cbt)


def _sc_gather(codebook, idx_flat):
    info = plsc.get_sparse_core_info()
    nc, ns = info.num_cores, info.num_subcores
    nw = nc * ns
    bpw = NTOK // nw
    mesh = plsc.VectorSubcoreMesh(core_axis_name="c", subcore_axis_name="s")

    @functools.partial(
        pl.kernel,
        mesh=mesh,
        out_type=jax.ShapeDtypeStruct((NTOK, EMB), F32),
        scratch_types=[
            pltpu.VMEM((bpw,), jnp.int32),
            pltpu.VMEM((bpw, EMB), F32),
            pltpu.SemaphoreType.DMA,
        ],
    )
    def gk(cb_hbm, idx_hbm, out_hbm, idx_v, rows_v, sem):
        wid = lax.axis_index("s") * nc + lax.axis_index("c")
        base = wid * bpw
        pltpu.sync_copy(idx_hbm.at[pl.ds(base, bpw)], idx_v)
        pltpu.async_copy(cb_hbm.at[idx_v], rows_v, sem).wait()
        pltpu.sync_copy(rows_v, out_hbm.at[pl.ds(base, bpw)])

    return gk(codebook, idx_flat)


def _dec_body(q_ref, z_ref, w1_ref, b1_ref, w2_ref, b2_ref, w3_ref, b3_ref,
              cnt_ref, out_ref, loss_ref, perp_ref, qp_ref, dp_ref, ep_ref,
              op_ref, acc_ref):
    pid = pl.program_id(0)
    qp_ref[0:1, :] = jnp.zeros((1, EMB), F32)
    qp_ref[T + 1:T + 2, :] = jnp.zeros((1, EMB), F32)
    qp_ref[1:T + 1, :] = q_ref[0]
    d1 = b1_ref[0, :][None, :]
    for j in range(3):
        d1 = d1 + _mm(qp_ref[j:T + j, :], w1_ref[j])
    d1 = jnp.maximum(d1, 0.0)
    zrow = jnp.zeros((1, HID), F32)
    dp_ref[0:1, :] = zrow
    dp_ref[T + 1:T + 2, :] = zrow
    dp_ref[1:T + 1, :] = d1
    b2 = b2_ref[0, :][None, :]
    ev = jnp.maximum(_mm(dp_ref[1:T + 1, :], w2_ref[1]) + _mm(dp_ref[0:T, :], w2_ref[3]) + b2, 0.0)
    od = jnp.maximum(_mm(dp_ref[2:T + 2, :], w2_ref[0]) + _mm(dp_ref[1:T + 1, :], w2_ref[2]) + b2, 0.0)
    ep_ref[0:1, :] = zrow
    ep_ref[T + 1:T + 2, :] = zrow
    ep_ref[1:T + 1, :] = ev
    op_ref[0:1, :] = zrow
    op_ref[T + 1:T + 2, :] = zrow
    op_ref[1:T + 1, :] = od
    b3 = b3_ref[0, :][None, :]
    out_ref[0, 0] = _mm(ep_ref[1:T + 1, :], w3_ref[1]) + _mm(op_ref[0:T, :], w3_ref[3]) + b3
    out_ref[0, 1] = _mm(op_ref[1:T + 1, :], w3_ref[0]) + _mm(ep_ref[1:T + 1, :], w3_ref[2]) + b3
    out_ref[0, 2] = _mm(op_ref[1:T + 1, :], w3_ref[1]) + _mm(ep_ref[1:T + 1, :], w3_ref[3]) + b3
    out_ref[0, 3] = _mm(ep_ref[2:T + 2, :], w3_ref[0]) + _mm(op_ref[1:T + 1, :], w3_ref[2]) + b3
    diff = q_ref[0] - z_ref[0]
    s = jnp.sum(diff * diff)

    @pl.when(pid == 0)
    def _():
        acc_ref[0, 0] = s

    @pl.when(pid != 0)
    def _():
        acc_ref[0, 0] = acc_ref[0, 0] + s

    @pl.when(pid == B - 1)
    def _():
        p = cnt_ref[0, :] * (1.0 / NTOK)
        ent = -jnp.sum(p * jnp.log(p + 1e-10))
        perp_ref[...] = jnp.full((1, 1), jnp.exp(ent), F32)
        loss_ref[...] = jnp.full((1, 1), 1.25 * acc_ref[0, 0] / (NTOK * EMB), F32)


def _decode(q, z, w1, b1, w2, b2, w3, b3, cnt):
    return pl.pallas_call(
        _dec_body,
        grid=(B,),
        in_specs=[
            pl.BlockSpec((1, T, EMB), lambda i: (i, 0, 0)),
            pl.BlockSpec((1, T, EMB), lambda i: (i, 0, 0)),
            pl.BlockSpec((3, EMB, HID), lambda i: (0, 0, 0)),
            pl.BlockSpec((1, HID), lambda i: (0, 0)),
            pl.BlockSpec((4, HID, HID), lambda i: (0, 0, 0)),
            pl.BlockSpec((1, HID), lambda i: (0, 0)),
            pl.BlockSpec((4, HID, IN_DIM), lambda i: (0, 0, 0)),
            pl.BlockSpec((1, IN_DIM), lambda i: (0, 0)),
            pl.BlockSpec((1, K), lambda i: (0, 0)),
        ],
        out_specs=[
            pl.BlockSpec((1, 4, T, IN_DIM), lambda i: (i, 0, 0, 0)),
            pl.BlockSpec((1, 1), lambda i: (0, 0)),
            pl.BlockSpec((1, 1), lambda i: (0, 0)),
        ],
        out_shape=[
            jax.ShapeDtypeStruct((B, 4, T, IN_DIM), F32),
            jax.ShapeDtypeStruct((1, 1), F32),
            jax.ShapeDtypeStruct((1, 1), F32),
        ],
        scratch_shapes=[
            pltpu.VMEM((T + 2, EMB), F32),
            pltpu.VMEM((T + 2, HID), F32),
            pltpu.VMEM((T + 2, HID), F32),
            pltpu.VMEM((T + 2, HID), F32),
            pltpu.SMEM((1, 1), F32),
        ],
    )(q, z, w1, b1, w2, b2, w3, b3, cnt)


def kernel(x, enc_w1, enc_b1, enc_w2, enc_b2, enc_w3, enc_b3, codebook,
           dec_w1, dec_b1, dec_w2, dec_b2, dec_w3, dec_b3):
    xpad = jnp.pad(jnp.transpose(x, (0, 2, 1)), ((0, 0), (1, 1), (0, 0)))
    xq0 = xpad[:, 0:L, :].reshape(B, T, 4 * IN_DIM)
    xq1 = xpad[:, 2:L + 2, :].reshape(B, T, 4 * IN_DIM)
    we1 = enc_w1.transpose(2, 1, 0).reshape(4 * IN_DIM, HID)
    we2 = enc_w2.transpose(2, 1, 0)
    we3 = enc_w3.transpose(2, 1, 0)

    z, idx3, counts = _encq(xq0, xq1, we1, enc_b1.reshape(1, HID), we2,
                            enc_b2.reshape(1, HID), we3, enc_b3.reshape(1, EMB),
                            codebook.T)

    q = _sc_gather(codebook, idx3.reshape(NTOK)).reshape(B, T, EMB)

    wd1 = dec_w1.transpose(2, 1, 0)
    wd2 = dec_w2.transpose(2, 0, 1)
    wd3 = dec_w3.transpose(2, 0, 1)
    streams, loss, perp = _decode(q, z, wd1, dec_b1.reshape(1, HID), wd2,
                                  dec_b2.reshape(1, HID), wd3,
                                  dec_b3.reshape(1, IN_DIM), counts)
    x_recon = streams.transpose(0, 3, 2, 1).reshape(B, IN_DIM, L)
    return (loss.reshape(()), x_recon, perp.reshape(()))


# speedup vs baseline: 2.1363x; 1.1176x over previous
import functools

import jax
import jax.numpy as jnp
from jax import lax
from jax.experimental import pallas as pl
from jax.experimental.pallas import tpu as pltpu
from jax.experimental.pallas import tpu_sc as plsc

B, IN_DIM, HID, EMB, K, L = 8, 128, 768, 256, 8192, 2048
T = L // 4
NTOK = B * T
F32 = jnp.float32
_KC = 2048


def _mm(a, b):
    return jnp.dot(a, b, preferred_element_type=F32)


def _mmt(a, b, precision=None):
    return lax.dot_general(a, b, (((1,), (1,)), ((), ())),
                           precision=precision, preferred_element_type=F32)


def _encq_body(xp_ref, w1_ref, b1_ref, w2_ref, b2_ref, w3_ref,
               b3_ref, cb_ref, z_ref, idx_ref, cnt_ref, ep_ref, op_ref, hp_ref):
    pid = pl.program_id(0)
    zrow = jnp.zeros((1, HID), F32)
    b1 = b1_ref[0, :][None, :]
    a0 = xp_ref[0, 0:L, :].reshape(T, 4 * IN_DIM)
    a1 = xp_ref[0, 2:L + 2, :].reshape(T, 4 * IN_DIM)
    he = jnp.maximum(_mm(a0, w1_ref[...]) + b1, 0.0)
    ho = jnp.maximum(_mm(a1, w1_ref[...]) + b1, 0.0)
    ep_ref[0:1, :] = zrow
    ep_ref[T + 1:T + 2, :] = zrow
    ep_ref[1:T + 1, :] = he
    op_ref[0:1, :] = zrow
    op_ref[T + 1:T + 2, :] = zrow
    op_ref[1:T + 1, :] = ho
    b2 = b2_ref[0, :][None, :]
    h2 = (_mm(op_ref[0:T, :], w2_ref[0]) + _mm(ep_ref[1:T + 1, :], w2_ref[1])
          + _mm(op_ref[1:T + 1, :], w2_ref[2]) + _mm(ep_ref[2:T + 2, :], w2_ref[3]) + b2)
    h2 = jnp.maximum(h2, 0.0)
    hp_ref[0:1, :] = zrow
    hp_ref[T + 1:T + 2, :] = zrow
    hp_ref[1:T + 1, :] = h2
    zb = b3_ref[0, :][None, :]
    for j in range(3):
        zb = zb + _mm(hp_ref[j:T + j, :], w3_ref[j])
    z_ref[0] = zb
    zn = jnp.sum(zb * zb, axis=1, keepdims=True)
    best_v = jnp.full((T, 1), jnp.inf, F32)
    best_i = jnp.zeros((T, 1), jnp.int32)
    for kc in range(K // _KC):
        cbt = cb_ref[:, kc * _KC:(kc + 1) * _KC]
        cn = jnp.sum(cbt * cbt, axis=0, keepdims=True)
        d = zn + cn - 2.0 * _mm(zb, cbt)
        m = jnp.min(d, axis=1, keepdims=True)
        ids = lax.broadcasted_iota(jnp.int32, (T, _KC), 1) + kc * _KC
        i = jnp.min(jnp.where(d == m, ids, K), axis=1, keepdims=True)
        upd = m < best_v
        best_i = jnp.where(upd, i, best_i)
        best_v = jnp.where(upd, m, best_v)
    idx_ref[0, 0, :] = best_i[:, 0]
    for kc in range(K // _KC):
        ids = lax.broadcasted_iota(jnp.int32, (1, _KC), 1) + kc * _KC
        cnt = jnp.sum((best_i == ids).astype(F32), axis=0)
        sl = pl.ds(kc * _KC, _KC)

        @pl.when(pid == 0)
        def _():
            cnt_ref[0, sl] = cnt

        @pl.when(pid != 0)
        def _():
            cnt_ref[0, sl] = cnt_ref[0, sl] + cnt


def _encq(xp, w1, b1, w2, b2, w3, b3, cb):
    return pl.pallas_call(
        _encq_body,
        grid=(B,),
        in_specs=[
            pl.BlockSpec((1, L + 2, IN_DIM), lambda i: (i, 0, 0)),
            pl.BlockSpec((4 * IN_DIM, HID), lambda i: (0, 0)),
            pl.BlockSpec((1, HID), lambda i: (0, 0)),
            pl.BlockSpec((4, HID, HID), lambda i: (0, 0, 0)),
            pl.BlockSpec((1, HID), lambda i: (0, 0)),
            pl.BlockSpec((3, HID, EMB), lambda i: (0, 0, 0)),
            pl.BlockSpec((1, EMB), lambda i: (0, 0)),
            pl.BlockSpec((EMB, K), lambda i: (0, 0)),
        ],
        out_specs=[
            pl.BlockSpec((1, T, EMB), lambda i: (i, 0, 0)),
            pl.BlockSpec((1, 1, T), lambda i: (i, 0, 0)),
            pl.BlockSpec((1, K), lambda i: (0, 0)),
        ],
        out_shape=[
            jax.ShapeDtypeStruct((B, T, EMB), F32),
            jax.ShapeDtypeStruct((B, 1, T), jnp.int32),
            jax.ShapeDtypeStruct((1, K), F32),
        ],
        scratch_shapes=[
            pltpu.VMEM((T + 2, HID), F32),
            pltpu.VMEM((T + 2, HID), F32),
            pltpu.VMEM((T + 2, HID), F32),
        ],
    )(xp, w1, b1, w2, b2, w3, b3, cb)


def _sc_gather(codebook, idx_flat):
    info = plsc.get_sparse_core_info()
    nc, ns = info.num_cores, info.num_subcores
    nw = nc * ns
    bpw = NTOK // nw
    mesh = plsc.VectorSubcoreMesh(core_axis_name="c", subcore_axis_name="s")

    @functools.partial(
        pl.kernel,
        mesh=mesh,
        out_type=jax.ShapeDtypeStruct((NTOK, EMB), F32),
        scratch_types=[
            pltpu.VMEM((bpw,), jnp.int32),
            pltpu.VMEM((bpw, EMB), F32),
            pltpu.SemaphoreType.DMA,
        ],
    )
    def gk(cb_hbm, idx_hbm, out_hbm, idx_v, rows_v, sem):
        wid = lax.axis_index("s") * nc + lax.axis_index("c")
        base = wid * bpw
        pltpu.sync_copy(idx_hbm.at[pl.ds(base, bpw)], idx_v)
        pltpu.async_copy(cb_hbm.at[idx_v], rows_v, sem).wait()
        pltpu.sync_copy(rows_v, out_hbm.at[pl.ds(base, bpw)])

    return gk(codebook, idx_flat)


def _dec_body(q_ref, z_ref, w1_ref, b1_ref, w2_ref, b2_ref, w3_ref, b3_ref,
              cnt_ref, out_ref, loss_ref, perp_ref, qp_ref, dp_ref, ep_ref,
              op_ref, acc_ref):
    pid = pl.program_id(0)
    qp_ref[0:1, :] = jnp.zeros((1, EMB), F32)
    qp_ref[T + 1:T + 2, :] = jnp.zeros((1, EMB), F32)
    qp_ref[1:T + 1, :] = q_ref[0]
    d1 = b1_ref[0, :][None, :]
    for j in range(3):
        d1 = d1 + _mm(qp_ref[j:T + j, :], w1_ref[j])
    d1 = jnp.maximum(d1, 0.0)
    zrow = jnp.zeros((1, HID), F32)
    dp_ref[0:1, :] = zrow
    dp_ref[T + 1:T + 2, :] = zrow
    dp_ref[1:T + 1, :] = d1
    b2 = b2_ref[0, :][None, :]
    ev = jnp.maximum(_mm(dp_ref[1:T + 1, :], w2_ref[1]) + _mm(dp_ref[0:T, :], w2_ref[3]) + b2, 0.0)
    od = jnp.maximum(_mm(dp_ref[2:T + 2, :], w2_ref[0]) + _mm(dp_ref[1:T + 1, :], w2_ref[2]) + b2, 0.0)
    ep_ref[0:1, :] = zrow
    ep_ref[T + 1:T + 2, :] = zrow
    ep_ref[1:T + 1, :] = ev
    op_ref[0:1, :] = zrow
    op_ref[T + 1:T + 2, :] = zrow
    op_ref[1:T + 1, :] = od
    b3 = b3_ref[0, :][None, :]
    out_ref[0, 0] = _mm(ep_ref[1:T + 1, :], w3_ref[1]) + _mm(op_ref[0:T, :], w3_ref[3]) + b3
    out_ref[0, 1] = _mm(op_ref[1:T + 1, :], w3_ref[0]) + _mm(ep_ref[1:T + 1, :], w3_ref[2]) + b3
    out_ref[0, 2] = _mm(op_ref[1:T + 1, :], w3_ref[1]) + _mm(ep_ref[1:T + 1, :], w3_ref[3]) + b3
    out_ref[0, 3] = _mm(ep_ref[2:T + 2, :], w3_ref[0]) + _mm(op_ref[1:T + 1, :], w3_ref[2]) + b3
    diff = q_ref[0] - z_ref[0]
    s = jnp.sum(diff * diff)

    @pl.when(pid == 0)
    def _():
        acc_ref[0, 0] = s

    @pl.when(pid != 0)
    def _():
        acc_ref[0, 0] = acc_ref[0, 0] + s

    @pl.when(pid == B - 1)
    def _():
        p = cnt_ref[0, :] * (1.0 / NTOK)
        ent = -jnp.sum(p * jnp.log(p + 1e-10))
        perp_ref[...] = jnp.full((1, 1), jnp.exp(ent), F32)
        loss_ref[...] = jnp.full((1, 1), 1.25 * acc_ref[0, 0] / (NTOK * EMB), F32)


def _decode(q, z, w1, b1, w2, b2, w3, b3, cnt):
    return pl.pallas_call(
        _dec_body,
        grid=(B,),
        in_specs=[
            pl.BlockSpec((1, T, EMB), lambda i: (i, 0, 0)),
            pl.BlockSpec((1, T, EMB), lambda i: (i, 0, 0)),
            pl.BlockSpec((3, EMB, HID), lambda i: (0, 0, 0)),
            pl.BlockSpec((1, HID), lambda i: (0, 0)),
            pl.BlockSpec((4, HID, HID), lambda i: (0, 0, 0)),
            pl.BlockSpec((1, HID), lambda i: (0, 0)),
            pl.BlockSpec((4, HID, IN_DIM), lambda i: (0, 0, 0)),
            pl.BlockSpec((1, IN_DIM), lambda i: (0, 0)),
            pl.BlockSpec((1, K), lambda i: (0, 0)),
        ],
        out_specs=[
            pl.BlockSpec((1, 4, T, IN_DIM), lambda i: (i, 0, 0, 0)),
            pl.BlockSpec((1, 1), lambda i: (0, 0)),
            pl.BlockSpec((1, 1), lambda i: (0, 0)),
        ],
        out_shape=[
            jax.ShapeDtypeStruct((B, 4, T, IN_DIM), F32),
            jax.ShapeDtypeStruct((1, 1), F32),
            jax.ShapeDtypeStruct((1, 1), F32),
        ],
        scratch_shapes=[
            pltpu.VMEM((T + 2, EMB), F32),
            pltpu.VMEM((T + 2, HID), F32),
            pltpu.VMEM((T + 2, HID), F32),
            pltpu.VMEM((T + 2, HID), F32),
            pltpu.SMEM((1, 1), F32),
        ],
    )(q, z, w1, b1, w2, b2, w3, b3, cnt)


def kernel(x, enc_w1, enc_b1, enc_w2, enc_b2, enc_w3, enc_b3, codebook,
           dec_w1, dec_b1, dec_w2, dec_b2, dec_w3, dec_b3):
    xpad = jnp.pad(jnp.transpose(x, (0, 2, 1)), ((0, 0), (1, 1), (0, 0)))
    we1 = enc_w1.transpose(2, 1, 0).reshape(4 * IN_DIM, HID)
    we2 = enc_w2.transpose(2, 1, 0)
    we3 = enc_w3.transpose(2, 1, 0)

    z, idx3, counts = _encq(xpad, we1, enc_b1.reshape(1, HID), we2,
                            enc_b2.reshape(1, HID), we3, enc_b3.reshape(1, EMB),
                            codebook.T)

    q = _sc_gather(codebook, idx3.reshape(NTOK)).reshape(B, T, EMB)

    wd1 = dec_w1.transpose(2, 1, 0)
    wd2 = dec_w2.transpose(2, 0, 1)
    wd3 = dec_w3.transpose(2, 0, 1)
    streams, loss, perp = _decode(q, z, wd1, dec_b1.reshape(1, HID), wd2,
                                  dec_b2.reshape(1, HID), wd3,
                                  dec_b3.reshape(1, IN_DIM), counts)
    x_recon = streams.transpose(0, 3, 2, 1).reshape(B, IN_DIM, L)
    return (loss.reshape(()), x_recon, perp.reshape(()))


# speedup vs baseline: 2.1852x; 1.0229x over previous
import functools

import jax
import jax.numpy as jnp
from jax import lax
from jax.experimental import pallas as pl
from jax.experimental.pallas import tpu as pltpu
from jax.experimental.pallas import tpu_sc as plsc

B, IN_DIM, HID, EMB, K, L = 8, 128, 768, 256, 8192, 2048
T = L // 4
NTOK = B * T
F32 = jnp.float32
_KC = 2048


def _mm(a, b):
    return jnp.dot(a, b, preferred_element_type=F32)


def _mmt(a, b, precision=None):
    return lax.dot_general(a, b, (((1,), (1,)), ((), ())),
                           precision=precision, preferred_element_type=F32)


def _encq_body(xp_ref, w1_ref, b1_ref, w2_ref, b2_ref, w3_ref,
               b3_ref, cb_ref, z_ref, idx_ref, cnt_ref, ep_ref, op_ref, hp_ref,
               cn_ref):
    pid = pl.program_id(0)
    zrow = jnp.zeros((1, HID), F32)
    b1 = b1_ref[0, :][None, :]
    a0 = xp_ref[0, 0:L, :].reshape(T, 4 * IN_DIM)
    a1 = xp_ref[0, 2:L + 2, :].reshape(T, 4 * IN_DIM)
    he = jnp.maximum(_mm(a0, w1_ref[...]) + b1, 0.0)
    ho = jnp.maximum(_mm(a1, w1_ref[...]) + b1, 0.0)
    ep_ref[0:1, :] = zrow
    ep_ref[T + 1:T + 2, :] = zrow
    ep_ref[1:T + 1, :] = he
    op_ref[0:1, :] = zrow
    op_ref[T + 1:T + 2, :] = zrow
    op_ref[1:T + 1, :] = ho
    b2 = b2_ref[0, :][None, :]
    h2 = (_mm(op_ref[0:T, :], w2_ref[0]) + _mm(ep_ref[1:T + 1, :], w2_ref[1])
          + _mm(op_ref[1:T + 1, :], w2_ref[2]) + _mm(ep_ref[2:T + 2, :], w2_ref[3]) + b2)
    h2 = jnp.maximum(h2, 0.0)
    hp_ref[0:1, :] = zrow
    hp_ref[T + 1:T + 2, :] = zrow
    hp_ref[1:T + 1, :] = h2
    zb = b3_ref[0, :][None, :]
    for j in range(3):
        zb = zb + _mm(hp_ref[j:T + j, :], w3_ref[j])
    z_ref[0] = zb
    zn = jnp.sum(zb * zb, axis=1, keepdims=True)

    @pl.when(pid == 0)
    def _():
        cbt_all = cb_ref[...]
        cn_ref[...] = 0.25 * jnp.sum(cbt_all * cbt_all, axis=0, keepdims=True)

    best_v = jnp.full((T, 1), jnp.inf, F32)
    best_i = jnp.zeros((T, 1), jnp.int32)
    for kc in range(K // _KC):
        cbt = cb_ref[:, kc * _KC:(kc + 1) * _KC]
        cn = cn_ref[0:1, kc * _KC:(kc + 1) * _KC]
        d = zn + cn + _mm(zb, cbt)
        m = jnp.min(d, axis=1, keepdims=True)
        ids = lax.broadcasted_iota(jnp.int32, (T, _KC), 1) + kc * _KC
        i = jnp.min(jnp.where(d == m, ids, K), axis=1, keepdims=True)
        upd = m < best_v
        best_i = jnp.where(upd, i, best_i)
        best_v = jnp.where(upd, m, best_v)
    idx_ref[0, 0, :] = best_i[:, 0]
    ones_col = jnp.ones((1, T), F32)
    for kc in range(K // _KC):
        ids = lax.broadcasted_iota(jnp.int32, (1, _KC), 1) + kc * _KC
        oh = (best_i == ids).astype(F32)
        cnt = _mm(ones_col, oh)[0, :]
        sl = pl.ds(kc * _KC, _KC)

        @pl.when(pid == 0)
        def _():
            cnt_ref[0, sl] = cnt

        @pl.when(pid != 0)
        def _():
            cnt_ref[0, sl] = cnt_ref[0, sl] + cnt


def _encq(xp, w1, b1, w2, b2, w3, b3, cb):
    return pl.pallas_call(
        _encq_body,
        grid=(B,),
        in_specs=[
            pl.BlockSpec((1, L + 2, IN_DIM), lambda i: (i, 0, 0)),
            pl.BlockSpec((4 * IN_DIM, HID), lambda i: (0, 0)),
            pl.BlockSpec((1, HID), lambda i: (0, 0)),
            pl.BlockSpec((4, HID, HID), lambda i: (0, 0, 0)),
            pl.BlockSpec((1, HID), lambda i: (0, 0)),
            pl.BlockSpec((3, HID, EMB), lambda i: (0, 0, 0)),
            pl.BlockSpec((1, EMB), lambda i: (0, 0)),
            pl.BlockSpec((EMB, K), lambda i: (0, 0)),
        ],
        out_specs=[
            pl.BlockSpec((1, T, EMB), lambda i: (i, 0, 0)),
            pl.BlockSpec((1, 1, T), lambda i: (i, 0, 0)),
            pl.BlockSpec((1, K), lambda i: (0, 0)),
        ],
        out_shape=[
            jax.ShapeDtypeStruct((B, T, EMB), F32),
            jax.ShapeDtypeStruct((B, 1, T), jnp.int32),
            jax.ShapeDtypeStruct((1, K), F32),
        ],
        scratch_shapes=[
            pltpu.VMEM((T + 2, HID), F32),
            pltpu.VMEM((T + 2, HID), F32),
            pltpu.VMEM((T + 2, HID), F32),
            pltpu.VMEM((1, K), F32),
        ],
    )(xp, w1, b1, w2, b2, w3, b3, cb)


def _sc_gather(codebook, idx_flat):
    info = plsc.get_sparse_core_info()
    nc, ns = info.num_cores, info.num_subcores
    nw = nc * ns
    bpw = NTOK // nw
    mesh = plsc.VectorSubcoreMesh(core_axis_name="c", subcore_axis_name="s")

    @functools.partial(
        pl.kernel,
        mesh=mesh,
        out_type=jax.ShapeDtypeStruct((NTOK, EMB), F32),
        scratch_types=[
            pltpu.VMEM((bpw,), jnp.int32),
            pltpu.VMEM((bpw, EMB), F32),
            pltpu.SemaphoreType.DMA,
        ],
    )
    def gk(cb_hbm, idx_hbm, out_hbm, idx_v, rows_v, sem):
        wid = lax.axis_index("s") * nc + lax.axis_index("c")
        base = wid * bpw
        pltpu.sync_copy(idx_hbm.at[pl.ds(base, bpw)], idx_v)
        pltpu.async_copy(cb_hbm.at[idx_v], rows_v, sem).wait()
        pltpu.sync_copy(rows_v, out_hbm.at[pl.ds(base, bpw)])

    return gk(codebook, idx_flat)


def _dec_body(q_ref, z_ref, w1_ref, b1_ref, w2_ref, b2_ref, w3_ref, b3_ref,
              cnt_ref, out_ref, loss_ref, perp_ref, qp_ref, dp_ref, ep_ref,
              op_ref, acc_ref):
    pid = pl.program_id(0)
    qp_ref[0:1, :] = jnp.zeros((1, EMB), F32)
    qp_ref[T + 1:T + 2, :] = jnp.zeros((1, EMB), F32)
    qp_ref[1:T + 1, :] = q_ref[0]
    d1 = b1_ref[0, :][None, :]
    for j in range(3):
        d1 = d1 + _mm(qp_ref[j:T + j, :], w1_ref[j])
    d1 = jnp.maximum(d1, 0.0)
    zrow = jnp.zeros((1, HID), F32)
    dp_ref[0:1, :] = zrow
    dp_ref[T + 1:T + 2, :] = zrow
    dp_ref[1:T + 1, :] = d1
    b2 = b2_ref[0, :][None, :]
    ev = jnp.maximum(_mm(dp_ref[1:T + 1, :], w2_ref[1]) + _mm(dp_ref[0:T, :], w2_ref[3]) + b2, 0.0)
    od = jnp.maximum(_mm(dp_ref[2:T + 2, :], w2_ref[0]) + _mm(dp_ref[1:T + 1, :], w2_ref[2]) + b2, 0.0)
    ep_ref[0:1, :] = zrow
    ep_ref[T + 1:T + 2, :] = zrow
    ep_ref[1:T + 1, :] = ev
    op_ref[0:1, :] = zrow
    op_ref[T + 1:T + 2, :] = zrow
    op_ref[1:T + 1, :] = od
    b3 = b3_ref[0, :][None, :]
    out_ref[0, 0] = _mm(ep_ref[1:T + 1, :], w3_ref[1]) + _mm(op_ref[0:T, :], w3_ref[3]) + b3
    out_ref[0, 1] = _mm(op_ref[1:T + 1, :], w3_ref[0]) + _mm(ep_ref[1:T + 1, :], w3_ref[2]) + b3
    out_ref[0, 2] = _mm(op_ref[1:T + 1, :], w3_ref[1]) + _mm(ep_ref[1:T + 1, :], w3_ref[3]) + b3
    out_ref[0, 3] = _mm(ep_ref[2:T + 2, :], w3_ref[0]) + _mm(op_ref[1:T + 1, :], w3_ref[2]) + b3
    diff = q_ref[0] - z_ref[0]
    s = jnp.sum(diff * diff)

    @pl.when(pid == 0)
    def _():
        acc_ref[0, 0] = s

    @pl.when(pid != 0)
    def _():
        acc_ref[0, 0] = acc_ref[0, 0] + s

    @pl.when(pid == B - 1)
    def _():
        p = cnt_ref[0, :] * (1.0 / NTOK)
        ent = -jnp.sum(p * jnp.log(p + 1e-10))
        perp_ref[...] = jnp.full((1, 1), jnp.exp(ent), F32)
        loss_ref[...] = jnp.full((1, 1), 1.25 * acc_ref[0, 0] / (NTOK * EMB), F32)


def _decode(q, z, w1, b1, w2, b2, w3, b3, cnt):
    return pl.pallas_call(
        _dec_body,
        grid=(B,),
        in_specs=[
            pl.BlockSpec((1, T, EMB), lambda i: (i, 0, 0)),
            pl.BlockSpec((1, T, EMB), lambda i: (i, 0, 0)),
            pl.BlockSpec((3, EMB, HID), lambda i: (0, 0, 0)),
            pl.BlockSpec((1, HID), lambda i: (0, 0)),
            pl.BlockSpec((4, HID, HID), lambda i: (0, 0, 0)),
            pl.BlockSpec((1, HID), lambda i: (0, 0)),
            pl.BlockSpec((4, HID, IN_DIM), lambda i: (0, 0, 0)),
            pl.BlockSpec((1, IN_DIM), lambda i: (0, 0)),
            pl.BlockSpec((1, K), lambda i: (0, 0)),
        ],
        out_specs=[
            pl.BlockSpec((1, 4, T, IN_DIM), lambda i: (i, 0, 0, 0)),
            pl.BlockSpec((1, 1), lambda i: (0, 0)),
            pl.BlockSpec((1, 1), lambda i: (0, 0)),
        ],
        out_shape=[
            jax.ShapeDtypeStruct((B, 4, T, IN_DIM), F32),
            jax.ShapeDtypeStruct((1, 1), F32),
            jax.ShapeDtypeStruct((1, 1), F32),
        ],
        scratch_shapes=[
            pltpu.VMEM((T + 2, EMB), F32),
            pltpu.VMEM((T + 2, HID), F32),
            pltpu.VMEM((T + 2, HID), F32),
            pltpu.VMEM((T + 2, HID), F32),
            pltpu.SMEM((1, 1), F32),
        ],
    )(q, z, w1, b1, w2, b2, w3, b3, cnt)


def kernel(x, enc_w1, enc_b1, enc_w2, enc_b2, enc_w3, enc_b3, codebook,
           dec_w1, dec_b1, dec_w2, dec_b2, dec_w3, dec_b3):
    xpad = jnp.pad(jnp.transpose(x, (0, 2, 1)), ((0, 0), (1, 1), (0, 0)))
    we1 = enc_w1.transpose(2, 1, 0).reshape(4 * IN_DIM, HID)
    we2 = enc_w2.transpose(2, 1, 0)
    we3 = enc_w3.transpose(2, 1, 0)

    z, idx3, counts = _encq(xpad, we1, enc_b1.reshape(1, HID), we2,
                            enc_b2.reshape(1, HID), we3, enc_b3.reshape(1, EMB),
                            -2.0 * codebook.T)

    q = _sc_gather(codebook, idx3.reshape(NTOK)).reshape(B, T, EMB)

    wd1 = dec_w1.transpose(2, 1, 0)
    wd2 = dec_w2.transpose(2, 0, 1)
    wd3 = dec_w3.transpose(2, 0, 1)
    streams, loss, perp = _decode(q, z, wd1, dec_b1.reshape(1, HID), wd2,
                                  dec_b2.reshape(1, HID), wd3,
                                  dec_b3.reshape(1, IN_DIM), counts)
    x_recon = streams.transpose(0, 3, 2, 1).reshape(B, IN_DIM, L)
    return (loss.reshape(()), x_recon, perp.reshape(()))


# speedup vs baseline: 2.1896x; 1.0020x over previous
import functools

import jax
import jax.numpy as jnp
from jax import lax
from jax.experimental import pallas as pl
from jax.experimental.pallas import tpu as pltpu
from jax.experimental.pallas import tpu_sc as plsc

B, IN_DIM, HID, EMB, K, L = 8, 128, 768, 256, 8192, 2048
T = L // 4
NTOK = B * T
F32 = jnp.float32
_KC = 2048


def _mm(a, b):
    return jnp.dot(a, b, preferred_element_type=F32)


def _mmt(a, b, precision=None):
    return lax.dot_general(a, b, (((1,), (1,)), ((), ())),
                           precision=precision, preferred_element_type=F32)


def _encq_body(xp_ref, w1_ref, b1_ref, w2_ref, b2_ref, w3_ref,
               b3_ref, cb_ref, idx_ref, loss_ref, perp_ref, ep_ref, op_ref,
               hp_ref, cn_ref, cnt_ref, acc_ref):
    pid = pl.program_id(0)
    zrow = jnp.zeros((1, HID), F32)
    b1 = b1_ref[0, :][None, :]
    a0 = xp_ref[0, 0:L, :].reshape(T, 4 * IN_DIM)
    a1 = xp_ref[0, 2:L + 2, :].reshape(T, 4 * IN_DIM)
    he = jnp.maximum(_mm(a0, w1_ref[...]) + b1, 0.0)
    ho = jnp.maximum(_mm(a1, w1_ref[...]) + b1, 0.0)
    ep_ref[0:1, :] = zrow
    ep_ref[T + 1:T + 2, :] = zrow
    ep_ref[1:T + 1, :] = he
    op_ref[0:1, :] = zrow
    op_ref[T + 1:T + 2, :] = zrow
    op_ref[1:T + 1, :] = ho
    b2 = b2_ref[0, :][None, :]
    h2 = (_mm(op_ref[0:T, :], w2_ref[0]) + _mm(ep_ref[1:T + 1, :], w2_ref[1])
          + _mm(op_ref[1:T + 1, :], w2_ref[2]) + _mm(ep_ref[2:T + 2, :], w2_ref[3]) + b2)
    h2 = jnp.maximum(h2, 0.0)
    hp_ref[0:1, :] = zrow
    hp_ref[T + 1:T + 2, :] = zrow
    hp_ref[1:T + 1, :] = h2
    zb = b3_ref[0, :][None, :]
    for j in range(3):
        zb = zb + _mm(hp_ref[j:T + j, :], w3_ref[j])
    zn = jnp.sum(zb * zb, axis=1, keepdims=True)

    @pl.when(pid == 0)
    def _():
        cbt_all = cb_ref[...]
        cn_ref[...] = 0.25 * jnp.sum(cbt_all * cbt_all, axis=0, keepdims=True)

    best_v = jnp.full((T, 1), jnp.inf, F32)
    best_i = jnp.zeros((T, 1), jnp.int32)
    for kc in range(K // _KC):
        cbt = cb_ref[:, kc * _KC:(kc + 1) * _KC]
        cn = cn_ref[0:1, kc * _KC:(kc + 1) * _KC]
        d = zn + cn + _mm(zb, cbt)
        m = jnp.min(d, axis=1, keepdims=True)
        ids = lax.broadcasted_iota(jnp.int32, (T, _KC), 1) + kc * _KC
        i = jnp.min(jnp.where(d == m, ids, K), axis=1, keepdims=True)
        upd = m < best_v
        best_i = jnp.where(upd, i, best_i)
        best_v = jnp.where(upd, m, best_v)
    idx_ref[0, 0, :] = best_i[:, 0]
    ones_col = jnp.ones((1, T), F32)
    for kc in range(K // _KC):
        ids = lax.broadcasted_iota(jnp.int32, (1, _KC), 1) + kc * _KC
        oh = (best_i == ids).astype(F32)
        cnt = _mm(ones_col, oh)[0, :]
        sl = pl.ds(kc * _KC, _KC)

        @pl.when(pid == 0)
        def _():
            cnt_ref[0, sl] = cnt

        @pl.when(pid != 0)
        def _():
            cnt_ref[0, sl] = cnt_ref[0, sl] + cnt
    s = jnp.sum(best_v)

    @pl.when(pid == 0)
    def _():
        acc_ref[0, 0] = s

    @pl.when(pid != 0)
    def _():
        acc_ref[0, 0] = acc_ref[0, 0] + s

    @pl.when(pid == B - 1)
    def _():
        p = cnt_ref[0, :] * (1.0 / NTOK)
        ent = -jnp.sum(p * jnp.log(p + 1e-10))
        perp_ref[...] = jnp.full((1, 1), jnp.exp(ent), F32)
        loss_ref[...] = jnp.full((1, 1), 1.25 * acc_ref[0, 0] / (NTOK * EMB), F32)


def _encq(xp, w1, b1, w2, b2, w3, b3, cb):
    return pl.pallas_call(
        _encq_body,
        grid=(B,),
        in_specs=[
            pl.BlockSpec((1, L + 2, IN_DIM), lambda i: (i, 0, 0)),
            pl.BlockSpec((4 * IN_DIM, HID), lambda i: (0, 0)),
            pl.BlockSpec((1, HID), lambda i: (0, 0)),
            pl.BlockSpec((4, HID, HID), lambda i: (0, 0, 0)),
            pl.BlockSpec((1, HID), lambda i: (0, 0)),
            pl.BlockSpec((3, HID, EMB), lambda i: (0, 0, 0)),
            pl.BlockSpec((1, EMB), lambda i: (0, 0)),
            pl.BlockSpec((EMB, K), lambda i: (0, 0)),
        ],
        out_specs=[
            pl.BlockSpec((1, 1, T), lambda i: (i, 0, 0)),
            pl.BlockSpec((1, 1), lambda i: (0, 0)),
            pl.BlockSpec((1, 1), lambda i: (0, 0)),
        ],
        out_shape=[
            jax.ShapeDtypeStruct((B, 1, T), jnp.int32),
            jax.ShapeDtypeStruct((1, 1), F32),
            jax.ShapeDtypeStruct((1, 1), F32),
        ],
        scratch_shapes=[
            pltpu.VMEM((T + 2, HID), F32),
            pltpu.VMEM((T + 2, HID), F32),
            pltpu.VMEM((T + 2, HID), F32),
            pltpu.VMEM((1, K), F32),
            pltpu.VMEM((1, K), F32),
            pltpu.SMEM((1, 1), F32),
        ],
    )(xp, w1, b1, w2, b2, w3, b3, cb)


def _sc_gather(codebook, idx_flat):
    info = plsc.get_sparse_core_info()
    nc, ns = info.num_cores, info.num_subcores
    nw = nc * ns
    bpw = NTOK // nw
    mesh = plsc.VectorSubcoreMesh(core_axis_name="c", subcore_axis_name="s")

    @functools.partial(
        pl.kernel,
        mesh=mesh,
        out_type=jax.ShapeDtypeStruct((NTOK, EMB), F32),
        scratch_types=[
            pltpu.VMEM((bpw,), jnp.int32),
            pltpu.VMEM((bpw, EMB), F32),
            pltpu.SemaphoreType.DMA,
        ],
    )
    def gk(cb_hbm, idx_hbm, out_hbm, idx_v, rows_v, sem):
        wid = lax.axis_index("s") * nc + lax.axis_index("c")
        base = wid * bpw
        pltpu.sync_copy(idx_hbm.at[pl.ds(base, bpw)], idx_v)
        pltpu.async_copy(cb_hbm.at[idx_v], rows_v, sem).wait()
        pltpu.sync_copy(rows_v, out_hbm.at[pl.ds(base, bpw)])

    return gk(codebook, idx_flat)


def _dec_body(q_ref, w1_ref, b1_ref, w2_ref, b2_ref, w3_ref, b3_ref,
              out_ref, qp_ref, dp_ref, ep_ref, op_ref):
    qp_ref[0:1, :] = jnp.zeros((1, EMB), F32)
    qp_ref[T + 1:T + 2, :] = jnp.zeros((1, EMB), F32)
    qp_ref[1:T + 1, :] = q_ref[0]
    d1 = b1_ref[0, :][None, :]
    for j in range(3):
        d1 = d1 + _mm(qp_ref[j:T + j, :], w1_ref[j])
    d1 = jnp.maximum(d1, 0.0)
    zrow = jnp.zeros((1, HID), F32)
    dp_ref[0:1, :] = zrow
    dp_ref[T + 1:T + 2, :] = zrow
    dp_ref[1:T + 1, :] = d1
    b2 = b2_ref[0, :][None, :]
    ev = jnp.maximum(_mm(dp_ref[1:T + 1, :], w2_ref[1]) + _mm(dp_ref[0:T, :], w2_ref[3]) + b2, 0.0)
    od = jnp.maximum(_mm(dp_ref[2:T + 2, :], w2_ref[0]) + _mm(dp_ref[1:T + 1, :], w2_ref[2]) + b2, 0.0)
    ep_ref[0:1, :] = zrow
    ep_ref[T + 1:T + 2, :] = zrow
    ep_ref[1:T + 1, :] = ev
    op_ref[0:1, :] = zrow
    op_ref[T + 1:T + 2, :] = zrow
    op_ref[1:T + 1, :] = od
    b3 = b3_ref[0, :][None, :]
    out_ref[0, 0] = _mm(ep_ref[1:T + 1, :], w3_ref[1]) + _mm(op_ref[0:T, :], w3_ref[3]) + b3
    out_ref[0, 1] = _mm(op_ref[1:T + 1, :], w3_ref[0]) + _mm(ep_ref[1:T + 1, :], w3_ref[2]) + b3
    out_ref[0, 2] = _mm(op_ref[1:T + 1, :], w3_ref[1]) + _mm(ep_ref[1:T + 1, :], w3_ref[3]) + b3
    out_ref[0, 3] = _mm(ep_ref[2:T + 2, :], w3_ref[0]) + _mm(op_ref[1:T + 1, :], w3_ref[2]) + b3


def _decode(q, w1, b1, w2, b2, w3, b3):
    return pl.pallas_call(
        _dec_body,
        grid=(B,),
        in_specs=[
            pl.BlockSpec((1, T, EMB), lambda i: (i, 0, 0)),
            pl.BlockSpec((3, EMB, HID), lambda i: (0, 0, 0)),
            pl.BlockSpec((1, HID), lambda i: (0, 0)),
            pl.BlockSpec((4, HID, HID), lambda i: (0, 0, 0)),
            pl.BlockSpec((1, HID), lambda i: (0, 0)),
            pl.BlockSpec((4, HID, IN_DIM), lambda i: (0, 0, 0)),
            pl.BlockSpec((1, IN_DIM), lambda i: (0, 0)),
        ],
        out_specs=pl.BlockSpec((1, 4, T, IN_DIM), lambda i: (i, 0, 0, 0)),
        out_shape=jax.ShapeDtypeStruct((B, 4, T, IN_DIM), F32),
        scratch_shapes=[
            pltpu.VMEM((T + 2, EMB), F32),
            pltpu.VMEM((T + 2, HID), F32),
            pltpu.VMEM((T + 2, HID), F32),
            pltpu.VMEM((T + 2, HID), F32),
        ],
    )(q, w1, b1, w2, b2, w3, b3)


def kernel(x, enc_w1, enc_b1, enc_w2, enc_b2, enc_w3, enc_b3, codebook,
           dec_w1, dec_b1, dec_w2, dec_b2, dec_w3, dec_b3):
    xpad = jnp.pad(jnp.transpose(x, (0, 2, 1)), ((0, 0), (1, 1), (0, 0)))
    we1 = enc_w1.transpose(2, 1, 0).reshape(4 * IN_DIM, HID)
    we2 = enc_w2.transpose(2, 1, 0)
    we3 = enc_w3.transpose(2, 1, 0)

    idx3, loss, perp = _encq(xpad, we1, enc_b1.reshape(1, HID), we2,
                             enc_b2.reshape(1, HID), we3, enc_b3.reshape(1, EMB),
                             -2.0 * codebook.T)

    q = _sc_gather(codebook, idx3.reshape(NTOK)).reshape(B, T, EMB)

    wd1 = dec_w1.transpose(2, 1, 0)
    wd2 = dec_w2.transpose(2, 0, 1)
    wd3 = dec_w3.transpose(2, 0, 1)
    streams = _decode(q, wd1, dec_b1.reshape(1, HID), wd2,
                      dec_b2.reshape(1, HID), wd3, dec_b3.reshape(1, IN_DIM))
    x_recon = streams.transpose(0, 3, 2, 1).reshape(B, IN_DIM, L)
    return (loss.reshape(()), x_recon, perp.reshape(()))
